# baseline (speedup 1.0000x reference)
import functools

import jax
import jax.numpy as jnp
from jax import lax
from jax.experimental import pallas as pl
from jax.experimental.pallas import tpu as pltpu
from jax.experimental.pallas import tpu_sc as plsc

NC = 2
NS = 16
NW = NC * NS
L = 16
CHUNK = 16
GATHER_W = 64


def _sc_body(nchunk, n_neg, dim,
             ug_h, uo_h, vpg_h, vpo_h, ngg_h, ngo_h, u_w_h, v_w_h,
             pos_out_h, neg_out_h,
             idx_ug, idx_uo, idx_vpg, idx_vpo, idx_ngg, idx_ngo,
             u_rows, vp_rows, ng_rows,
             pos_part, neg_part, sem):
  neg_g = CHUNK * n_neg // GATHER_W
  cid = lax.axis_index("c")
  sid = lax.axis_index("s")
  wid = sid * NC + cid

  pltpu.sync_copy(ug_h.at[wid], idx_ug)
  pltpu.sync_copy(uo_h.at[wid], idx_uo)
  pltpu.sync_copy(vpg_h.at[wid], idx_vpg)
  pltpu.sync_copy(vpo_h.at[wid], idx_vpo)
  pltpu.sync_copy(ngg_h.at[wid], idx_ngg)
  pltpu.sync_copy(ngo_h.at[wid], idx_ngo)

  lanes = jnp.arange(L, dtype=jnp.int32)

  def chunk_body(c, carry):
    cp_u = pltpu.async_copy(u_w_h.at[idx_ug.at[c]], u_rows, sem)
    cp_vp = pltpu.async_copy(v_w_h.at[idx_vpg.at[c]], vp_rows, sem)
    cps = [
        pltpu.async_copy(v_w_h.at[idx_ngg.at[c, g]], ng_rows.at[g], sem)
        for g in range(neg_g)
    ]
    cp_u.wait()
    cp_vp.wait()
    for cp in cps:
      cp.wait()

    off_u = idx_uo[c, :]
    off_vp = idx_vpo[c, :]
    off_ng = [idx_ngo[c, n, :] for n in range(n_neg)]
    ng_blk = [jnp.full((L,), (n * L) // GATHER_W, jnp.int32)
              for n in range(n_neg)]
    ng_row = [lanes + (n * L) % GATHER_W for n in range(n_neg)]

    def d_body(d, accs):
      pos_acc, neg_acc = accs
      dv = jnp.full((L,), d, jnp.int32)
      u_d = plsc.load_gather(u_rows, [lanes, off_u + dv])
      vp_d = plsc.load_gather(vp_rows, [lanes, off_vp + dv])
      s = plsc.load_gather(ng_rows, [ng_blk[0], ng_row[0], off_ng[0] + dv])
      for n in range(1, n_neg):
        s = s + plsc.load_gather(
            ng_rows, [ng_blk[n], ng_row[n], off_ng[n] + dv])
      return pos_acc + u_d * vp_d, neg_acc + u_d * s

    zero = jnp.zeros((L,), jnp.float32)
    pos_acc, neg_acc = lax.fori_loop(0, dim, d_body, (zero, zero))
    pos_part[c, :] = pos_acc
    neg_part[c, :] = neg_acc
    return carry

  lax.fori_loop(0, nchunk, chunk_body, 0)

  pltpu.sync_copy(pos_part, pos_out_h.at[wid])
  pltpu.sync_copy(neg_part, neg_out_h.at[wid])


def _tc_body(pos_ref, neg_ref, out_ref):
  cost = jax.nn.log_sigmoid(pos_ref[...]) + jax.nn.log_sigmoid(-neg_ref[...])
  out_ref[...] = jnp.sum(cost).reshape(1, 1)


def _pair_table(w):
  return jnp.concatenate([w[0::2, :], w[1::2, :]], axis=1)


@jax.jit
def kernel(u_pos, v_pos, v_neg, batch_size, u_weight, v_weight):
  b = u_pos.shape[0]
  n_neg = v_neg.shape[1]
  vocab, dim = u_weight.shape
  pw = 2 * dim
  bpw = b // NW
  nchunk = bpw // CHUNK
  neg_g = CHUNK * n_neg // GATHER_W

  uw2 = _pair_table(u_weight)
  vw2 = _pair_table(v_weight)

  u_pos = u_pos.astype(jnp.int32)
  v_pos = v_pos.astype(jnp.int32)
  v_neg = v_neg.astype(jnp.int32)
  ug = (u_pos >> 1).reshape(NW, nchunk, CHUNK)
  uo = ((u_pos & 1) * dim).reshape(NW, nchunk, CHUNK)
  vpg = (v_pos >> 1).reshape(NW, nchunk, CHUNK)
  vpo = ((v_pos & 1) * dim).reshape(NW, nchunk, CHUNK)
  ngt = v_neg.reshape(NW, nchunk, CHUNK, n_neg).transpose(0, 1, 3, 2)
  ngg = (ngt >> 1).reshape(NW, nchunk, neg_g, GATHER_W)
  ngo = ((ngt & 1) * dim)

  mesh = plsc.VectorSubcoreMesh(
      core_axis_name="c", subcore_axis_name="s",
      num_cores=NC, num_subcores=NS)
  score_ty = jax.ShapeDtypeStruct((NW, nchunk, CHUNK), jnp.float32)
  sc = pl.kernel(
      functools.partial(_sc_body, nchunk, n_neg, dim),
      out_type=(score_ty, score_ty),
      mesh=mesh,
      compiler_params=pltpu.CompilerParams(
          use_tc_tiling_on_sc=False, needs_layout_passes=False),
      scratch_types=[
          pltpu.VMEM((nchunk, CHUNK), jnp.int32),
          pltpu.VMEM((nchunk, CHUNK), jnp.int32),
          pltpu.VMEM((nchunk, CHUNK), jnp.int32),
          pltpu.VMEM((nchunk, CHUNK), jnp.int32),
          pltpu.VMEM((nchunk, neg_g, GATHER_W), jnp.int32),
          pltpu.VMEM((nchunk, n_neg, CHUNK), jnp.int32),
          pltpu.VMEM((CHUNK, pw), jnp.float32),
          pltpu.VMEM((CHUNK, pw), jnp.float32),
          pltpu.VMEM((neg_g, GATHER_W, pw), jnp.float32),
          pltpu.VMEM((nchunk, CHUNK), jnp.float32),
          pltpu.VMEM((nchunk, CHUNK), jnp.float32),
          pltpu.SemaphoreType.DMA,
      ],
  )
  pos_s, neg_s = sc(ug, uo, vpg, vpo, ngg, ngo, uw2, vw2)

  rows = 128
  total = pl.pallas_call(
      _tc_body,
      in_specs=[
          pl.BlockSpec((rows, b // rows), lambda: (0, 0)),
          pl.BlockSpec((rows, b // rows), lambda: (0, 0)),
      ],
      out_specs=pl.BlockSpec((1, 1), lambda: (0, 0)),
      out_shape=jax.ShapeDtypeStruct((1, 1), jnp.float32),
  )(pos_s.reshape(rows, b // rows), neg_s.reshape(rows, b // rows))

  return -total[0, 0] / batch_size


# speedup vs baseline: 9.5870x; 9.5870x over previous
import functools

import jax
import jax.numpy as jnp
from jax import lax
from jax.experimental import pallas as pl
from jax.experimental.pallas import tpu as pltpu
from jax.experimental.pallas import tpu_sc as plsc

NC = 2
NS = 16
NW = NC * NS
L = 16
CHUNK = 16
GATHER_W = 64


def _sc_body(nchunk, n_neg, dim,
             ug_h, uo_h, vpg_h, vpo_h, ngg_h, ngo_h, u_w_h, v_w_h,
             pos_out_h, neg_out_h,
             idx_ug, idx_uo, idx_vpg, idx_vpo, idx_ngg, idx_ngo,
             u_rows, vp_rows, ng_rows,
             pos_part, neg_part, sem):
  neg_g = CHUNK * n_neg // GATHER_W
  cid = lax.axis_index("c")
  sid = lax.axis_index("s")
  wid = sid * NC + cid

  pltpu.sync_copy(ug_h.at[wid], idx_ug)
  pltpu.sync_copy(uo_h.at[wid], idx_uo)
  pltpu.sync_copy(vpg_h.at[wid], idx_vpg)
  pltpu.sync_copy(vpo_h.at[wid], idx_vpo)
  pltpu.sync_copy(ngg_h.at[wid], idx_ngg)
  pltpu.sync_copy(ngo_h.at[wid], idx_ngo)

  lanes = jnp.arange(L, dtype=jnp.int32)

  def chunk_body(c, carry):
    cp_u = pltpu.async_copy(u_w_h.at[idx_ug.at[c]], u_rows, sem)
    cp_vp = pltpu.async_copy(v_w_h.at[idx_vpg.at[c]], vp_rows, sem)
    cps = [
        pltpu.async_copy(v_w_h.at[idx_ngg.at[c, g]], ng_rows.at[g], sem)
        for g in range(neg_g)
    ]
    cp_u.wait()
    cp_vp.wait()
    for cp in cps:
      cp.wait()

    off_u = idx_uo[c, :]
    off_vp = idx_vpo[c, :]
    off_ng = [idx_ngo[c, n, :] for n in range(n_neg)]
    ng_blk = [jnp.full((L,), (n * L) // GATHER_W, jnp.int32)
              for n in range(n_neg)]
    ng_row = [lanes + (n * L) % GATHER_W for n in range(n_neg)]

    def d_body(d, accs):
      pos_acc, neg_acc = accs
      dv = jnp.full((L,), d, jnp.int32)
      u_d = plsc.load_gather(u_rows, [lanes, off_u + dv])
      vp_d = plsc.load_gather(vp_rows, [lanes, off_vp + dv])
      s = plsc.load_gather(ng_rows, [ng_blk[0], ng_row[0], off_ng[0] + dv])
      for n in range(1, n_neg):
        s = s + plsc.load_gather(
            ng_rows, [ng_blk[n], ng_row[n], off_ng[n] + dv])
      return pos_acc + u_d * vp_d, neg_acc + u_d * s

    zero = jnp.zeros((L,), jnp.float32)
    pos_acc, neg_acc = lax.fori_loop(0, dim, d_body, (zero, zero))
    pos_part[c, :] = pos_acc
    neg_part[c, :] = neg_acc
    return carry

  lax.fori_loop(0, nchunk, chunk_body, 0)

  pltpu.sync_copy(pos_part, pos_out_h.at[wid])
  pltpu.sync_copy(neg_part, neg_out_h.at[wid])


def _tc_body(pos_ref, neg_ref, out_ref):
  cost = jax.nn.log_sigmoid(pos_ref[...]) + jax.nn.log_sigmoid(-neg_ref[...])
  out_ref[...] = jnp.sum(cost).reshape(1, 1)


def _repack_body(in_ref, out_ref):
  x = in_ref[...]
  a = x.reshape(x.shape[0] // 2, 2, x.shape[1])
  out_ref[...] = jnp.concatenate([a[:, 0, :], a[:, 1, :]], axis=1)


def _pair_table(w):
  v, d = w.shape
  rb = 8000
  return pl.pallas_call(
      _repack_body,
      grid=(v // rb,),
      in_specs=[pl.BlockSpec((rb, d), lambda i: (i, 0))],
      out_specs=pl.BlockSpec((rb // 2, 2 * d), lambda i: (i, 0)),
      out_shape=jax.ShapeDtypeStruct((v // 2, 2 * d), jnp.float32),
  )(w)


@jax.jit
def kernel(u_pos, v_pos, v_neg, batch_size, u_weight, v_weight):
  b = u_pos.shape[0]
  n_neg = v_neg.shape[1]
  vocab, dim = u_weight.shape
  pw = 2 * dim
  bpw = b // NW
  nchunk = bpw // CHUNK
  neg_g = CHUNK * n_neg // GATHER_W

  uw2 = _pair_table(u_weight)
  vw2 = _pair_table(v_weight)

  u_pos = u_pos.astype(jnp.int32)
  v_pos = v_pos.astype(jnp.int32)
  v_neg = v_neg.astype(jnp.int32)
  ug = (u_pos >> 1).reshape(NW, nchunk, CHUNK)
  uo = ((u_pos & 1) * dim).reshape(NW, nchunk, CHUNK)
  vpg = (v_pos >> 1).reshape(NW, nchunk, CHUNK)
  vpo = ((v_pos & 1) * dim).reshape(NW, nchunk, CHUNK)
  ngt = v_neg.reshape(NW, nchunk, CHUNK, n_neg).transpose(0, 1, 3, 2)
  ngg = (ngt >> 1).reshape(NW, nchunk, neg_g, GATHER_W)
  ngo = ((ngt & 1) * dim)

  mesh = plsc.VectorSubcoreMesh(
      core_axis_name="c", subcore_axis_name="s",
      num_cores=NC, num_subcores=NS)
  score_ty = jax.ShapeDtypeStruct((NW, nchunk, CHUNK), jnp.float32)
  sc = pl.kernel(
      functools.partial(_sc_body, nchunk, n_neg, dim),
      out_type=(score_ty, score_ty),
      mesh=mesh,
      compiler_params=pltpu.CompilerParams(
          use_tc_tiling_on_sc=False, needs_layout_passes=False),
      scratch_types=[
          pltpu.VMEM((nchunk, CHUNK), jnp.int32),
          pltpu.VMEM((nchunk, CHUNK), jnp.int32),
          pltpu.VMEM((nchunk, CHUNK), jnp.int32),
          pltpu.VMEM((nchunk, CHUNK), jnp.int32),
          pltpu.VMEM((nchunk, neg_g, GATHER_W), jnp.int32),
          pltpu.VMEM((nchunk, n_neg, CHUNK), jnp.int32),
          pltpu.VMEM((CHUNK, pw), jnp.float32),
          pltpu.VMEM((CHUNK, pw), jnp.float32),
          pltpu.VMEM((neg_g, GATHER_W, pw), jnp.float32),
          pltpu.VMEM((nchunk, CHUNK), jnp.float32),
          pltpu.VMEM((nchunk, CHUNK), jnp.float32),
          pltpu.SemaphoreType.DMA,
      ],
  )
  pos_s, neg_s = sc(ug, uo, vpg, vpo, ngg, ngo, uw2, vw2)

  rows = 128
  total = pl.pallas_call(
      _tc_body,
      in_specs=[
          pl.BlockSpec((rows, b // rows), lambda: (0, 0)),
          pl.BlockSpec((rows, b // rows), lambda: (0, 0)),
      ],
      out_specs=pl.BlockSpec((1, 1), lambda: (0, 0)),
      out_shape=jax.ShapeDtypeStruct((1, 1), jnp.float32),
  )(pos_s.reshape(rows, b // rows), neg_s.reshape(rows, b // rows))

  return -total[0, 0] / batch_size


# speedup vs baseline: 10.9779x; 1.1451x over previous
import functools

import jax
import jax.numpy as jnp
from jax import lax
from jax.experimental import pallas as pl
from jax.experimental.pallas import tpu as pltpu
from jax.experimental.pallas import tpu_sc as plsc

NC = 2
NS = 16
NW = NC * NS
L = 16
CHUNK = 16
GATHER_W = 64


def _sc_body(nchunk, n_neg, dim,
             ug_h, uo_h, vpg_h, vpo_h, ngg_h, ngo_h, u_w_h, v_w_h,
             pos_out_h, neg_out_h,
             idx_ug, idx_uo, idx_vpg, idx_vpo, idx_ngg, idx_ngo,
             u_rows, vp_rows, ng_rows,
             pos_part, neg_part, sem):
  neg_g = CHUNK * n_neg // GATHER_W
  cid = lax.axis_index("c")
  sid = lax.axis_index("s")
  wid = sid * NC + cid

  pltpu.sync_copy(ug_h.at[wid], idx_ug)
  pltpu.sync_copy(uo_h.at[wid], idx_uo)
  pltpu.sync_copy(vpg_h.at[wid], idx_vpg)
  pltpu.sync_copy(vpo_h.at[wid], idx_vpo)
  pltpu.sync_copy(ngg_h.at[wid], idx_ngg)
  pltpu.sync_copy(ngo_h.at[wid], idx_ngo)

  lanes = jnp.arange(L, dtype=jnp.int32)

  def chunk_body(c, carry):
    cp_u = pltpu.async_copy(u_w_h.at[idx_ug.at[c]], u_rows, sem)
    cp_vp = pltpu.async_copy(v_w_h.at[idx_vpg.at[c]], vp_rows, sem)
    cps = [
        pltpu.async_copy(v_w_h.at[idx_ngg.at[c, g]], ng_rows.at[g], sem)
        for g in range(neg_g)
    ]
    cp_u.wait()
    cp_vp.wait()
    for cp in cps:
      cp.wait()

    off_u = idx_uo[c, :]
    off_vp = idx_vpo[c, :]
    off_ng = [idx_ngo[c, n, :] for n in range(n_neg)]
    ng_blk = [jnp.full((L,), (n * L) // GATHER_W, jnp.int32)
              for n in range(n_neg)]
    ng_row = [lanes + (n * L) % GATHER_W for n in range(n_neg)]

    def d_body(d, accs):
      pos_acc, neg_acc = accs
      dv = jnp.full((L,), d, jnp.int32)
      u_d = plsc.load_gather(u_rows, [lanes, off_u + dv])
      vp_d = plsc.load_gather(vp_rows, [lanes, off_vp + dv])
      s = plsc.load_gather(ng_rows, [ng_blk[0], ng_row[0], off_ng[0] + dv])
      for n in range(1, n_neg):
        s = s + plsc.load_gather(
            ng_rows, [ng_blk[n], ng_row[n], off_ng[n] + dv])
      return pos_acc + u_d * vp_d, neg_acc + u_d * s

    zero = jnp.zeros((L,), jnp.float32)
    pos_acc, neg_acc = lax.fori_loop(0, dim, d_body, (zero, zero))
    pos_part[c, :] = pos_acc
    neg_part[c, :] = neg_acc
    return carry

  lax.fori_loop(0, nchunk, chunk_body, 0)

  pltpu.sync_copy(pos_part, pos_out_h.at[wid])
  pltpu.sync_copy(neg_part, neg_out_h.at[wid])


def _tc_body(pos_ref, neg_ref, out_ref):
  cost = jax.nn.log_sigmoid(pos_ref[...]) + jax.nn.log_sigmoid(-neg_ref[...])
  out_ref[...] = jnp.sum(cost).reshape(1, 1)


def _repack_body(a_ref, b_ref, out_ref):
  out_ref[...] = jnp.concatenate([a_ref[...], b_ref[...]], axis=1)


def _pair_table(w):
  v, d = w.shape
  h = v // 2
  rb = 4000
  nblk = h // rb
  return pl.pallas_call(
      _repack_body,
      grid=(nblk,),
      in_specs=[
          pl.BlockSpec((rb, d), lambda i: (i, 0)),
          pl.BlockSpec((rb, d), lambda i, _n=nblk: (i + _n, 0)),
      ],
      out_specs=pl.BlockSpec((rb, 2 * d), lambda i: (i, 0)),
      out_shape=jax.ShapeDtypeStruct((h, 2 * d), jnp.float32),
  )(w, w)


@jax.jit
def kernel(u_pos, v_pos, v_neg, batch_size, u_weight, v_weight):
  b = u_pos.shape[0]
  n_neg = v_neg.shape[1]
  vocab, dim = u_weight.shape
  pw = 2 * dim
  bpw = b // NW
  nchunk = bpw // CHUNK
  neg_g = CHUNK * n_neg // GATHER_W

  uw2 = _pair_table(u_weight)
  vw2 = _pair_table(v_weight)

  half = vocab // 2
  u_pos = u_pos.astype(jnp.int32)
  v_pos = v_pos.astype(jnp.int32)
  v_neg = v_neg.astype(jnp.int32)
  ug = (u_pos % half).reshape(NW, nchunk, CHUNK)
  uo = ((u_pos >= half) * dim).reshape(NW, nchunk, CHUNK)
  vpg = (v_pos % half).reshape(NW, nchunk, CHUNK)
  vpo = ((v_pos >= half) * dim).reshape(NW, nchunk, CHUNK)
  ngt = v_neg.reshape(NW, nchunk, CHUNK, n_neg).transpose(0, 1, 3, 2)
  ngg = (ngt % half).reshape(NW, nchunk, neg_g, GATHER_W)
  ngo = ((ngt >= half) * dim)

  mesh = plsc.VectorSubcoreMesh(
      core_axis_name="c", subcore_axis_name="s",
      num_cores=NC, num_subcores=NS)
  score_ty = jax.ShapeDtypeStruct((NW, nchunk, CHUNK), jnp.float32)
  sc = pl.kernel(
      functools.partial(_sc_body, nchunk, n_neg, dim),
      out_type=(score_ty, score_ty),
      mesh=mesh,
      compiler_params=pltpu.CompilerParams(
          use_tc_tiling_on_sc=False, needs_layout_passes=False),
      scratch_types=[
          pltpu.VMEM((nchunk, CHUNK), jnp.int32),
          pltpu.VMEM((nchunk, CHUNK), jnp.int32),
          pltpu.VMEM((nchunk, CHUNK), jnp.int32),
          pltpu.VMEM((nchunk, CHUNK), jnp.int32),
          pltpu.VMEM((nchunk, neg_g, GATHER_W), jnp.int32),
          pltpu.VMEM((nchunk, n_neg, CHUNK), jnp.int32),
          pltpu.VMEM((CHUNK, pw), jnp.float32),
          pltpu.VMEM((CHUNK, pw), jnp.float32),
          pltpu.VMEM((neg_g, GATHER_W, pw), jnp.float32),
          pltpu.VMEM((nchunk, CHUNK), jnp.float32),
          pltpu.VMEM((nchunk, CHUNK), jnp.float32),
          pltpu.SemaphoreType.DMA,
      ],
  )
  pos_s, neg_s = sc(ug, uo, vpg, vpo, ngg, ngo, uw2, vw2)

  rows = 128
  total = pl.pallas_call(
      _tc_body,
      in_specs=[
          pl.BlockSpec((rows, b // rows), lambda: (0, 0)),
          pl.BlockSpec((rows, b // rows), lambda: (0, 0)),
      ],
      out_specs=pl.BlockSpec((1, 1), lambda: (0, 0)),
      out_shape=jax.ShapeDtypeStruct((1, 1), jnp.float32),
  )(pos_s.reshape(rows, b // rows), neg_s.reshape(rows, b // rows))

  return -total[0, 0] / batch_size


# speedup vs baseline: 12.2705x; 1.1177x over previous
import functools

import jax
import jax.numpy as jnp
from jax import lax
from jax.experimental import pallas as pl
from jax.experimental.pallas import tpu as pltpu
from jax.experimental.pallas import tpu_sc as plsc

NC = 2
NS = 16
NW = NC * NS
L = 16
CHUNK = 16
GATHER_W = 64


def _sc_body(nchunk, n_neg, dim,
             ug_h, uo_h, vpg_h, vpo_h, ngg_h, ngo_h, u_w_h, v_w_h,
             pos_out_h, neg_out_h,
             idx_ug, idx_uo, idx_vpg, idx_vpo, idx_ngg, idx_ngo,
             u_rows, vp_rows, ng_rows,
             pos_part, neg_part, sem):
  neg_g = CHUNK * n_neg // GATHER_W
  cid = lax.axis_index("c")
  sid = lax.axis_index("s")
  wid = sid * NC + cid

  pltpu.sync_copy(ug_h.at[wid], idx_ug)
  pltpu.sync_copy(uo_h.at[wid], idx_uo)
  pltpu.sync_copy(vpg_h.at[wid], idx_vpg)
  pltpu.sync_copy(vpo_h.at[wid], idx_vpo)
  pltpu.sync_copy(ngg_h.at[wid], idx_ngg)
  pltpu.sync_copy(ngo_h.at[wid], idx_ngo)

  lanes = jnp.arange(L, dtype=jnp.int32)

  def chunk_body(c, carry):
    cp_u = pltpu.async_copy(u_w_h.at[idx_ug.at[c]], u_rows, sem)
    cp_vp = pltpu.async_copy(v_w_h.at[idx_vpg.at[c]], vp_rows, sem)
    cps = [
        pltpu.async_copy(v_w_h.at[idx_ngg.at[c, g]], ng_rows.at[g], sem)
        for g in range(neg_g)
    ]
    cp_u.wait()
    cp_vp.wait()
    for cp in cps:
      cp.wait()

    off_u = idx_uo[c, :]
    off_vp = idx_vpo[c, :]
    off_ng = [idx_ngo[c, n, :] for n in range(n_neg)]
    ng_blk = [jnp.full((L,), (n * L) // GATHER_W, jnp.int32)
              for n in range(n_neg)]
    ng_row = [lanes + (n * L) % GATHER_W for n in range(n_neg)]

    def d_body(d, accs):
      pos_acc, neg_acc = accs
      dv = jnp.full((L,), d, jnp.int32)
      u_d = plsc.load_gather(u_rows, [lanes, off_u + dv])
      vp_d = plsc.load_gather(vp_rows, [lanes, off_vp + dv])
      s = plsc.load_gather(ng_rows, [ng_blk[0], ng_row[0], off_ng[0] + dv])
      for n in range(1, n_neg):
        s = s + plsc.load_gather(
            ng_rows, [ng_blk[n], ng_row[n], off_ng[n] + dv])
      return pos_acc + u_d * vp_d, neg_acc + u_d * s

    zero = jnp.zeros((L,), jnp.float32)
    pos_acc, neg_acc = lax.fori_loop(0, dim, d_body, (zero, zero))
    pos_part[c, :] = pos_acc
    neg_part[c, :] = neg_acc
    return carry

  lax.fori_loop(0, nchunk, chunk_body, 0)

  pltpu.sync_copy(pos_part, pos_out_h.at[wid])
  pltpu.sync_copy(neg_part, neg_out_h.at[wid])


def _tc_body(pos_ref, neg_ref, out_ref):
  cost = jax.nn.log_sigmoid(pos_ref[...]) + jax.nn.log_sigmoid(-neg_ref[...])
  out_ref[...] = jnp.sum(cost).reshape(1, 1)


def _repack_body(a_ref, b_ref, out_ref):
  out_ref[...] = jnp.concatenate([a_ref[...], b_ref[...]], axis=1)


def _pair_table(w):
  v, d = w.shape
  h = v // 2
  rb = 4000
  nblk = h // rb
  return pl.pallas_call(
      _repack_body,
      grid=(nblk,),
      in_specs=[
          pl.BlockSpec((rb, d), lambda i: (i, 0)),
          pl.BlockSpec((rb, d), lambda i, _n=nblk: (i + _n, 0)),
      ],
      out_specs=pl.BlockSpec((rb, 2 * d), lambda i: (i, 0)),
      out_shape=jax.ShapeDtypeStruct((h, 2 * d), jnp.float32),
  )(w, w)


@jax.jit
def kernel(u_pos, v_pos, v_neg, batch_size, u_weight, v_weight):
  b = u_pos.shape[0]
  n_neg = v_neg.shape[1]
  vocab, dim = u_weight.shape
  bpw = b // NW
  nchunk = bpw // CHUNK
  neg_g = CHUNK * n_neg // GATHER_W

  uw2 = u_weight
  vw2 = v_weight
  pw = dim

  u_pos = u_pos.astype(jnp.int32)
  v_pos = v_pos.astype(jnp.int32)
  v_neg = v_neg.astype(jnp.int32)
  ug = u_pos.reshape(NW, nchunk, CHUNK)
  uo = jnp.zeros((NW, nchunk, CHUNK), jnp.int32)
  vpg = v_pos.reshape(NW, nchunk, CHUNK)
  vpo = jnp.zeros((NW, nchunk, CHUNK), jnp.int32)
  ngt = v_neg.reshape(NW, nchunk, CHUNK, n_neg).transpose(0, 1, 3, 2)
  ngg = ngt.reshape(NW, nchunk, neg_g, GATHER_W)
  ngo = jnp.zeros((NW, nchunk, n_neg, CHUNK), jnp.int32)

  mesh = plsc.VectorSubcoreMesh(
      core_axis_name="c", subcore_axis_name="s",
      num_cores=NC, num_subcores=NS)
  score_ty = jax.ShapeDtypeStruct((NW, nchunk, CHUNK), jnp.float32)
  sc = pl.kernel(
      functools.partial(_sc_body, nchunk, n_neg, dim),
      out_type=(score_ty, score_ty),
      mesh=mesh,
      compiler_params=pltpu.CompilerParams(
          use_tc_tiling_on_sc=False, needs_layout_passes=False),
      scratch_types=[
          pltpu.VMEM((nchunk, CHUNK), jnp.int32),
          pltpu.VMEM((nchunk, CHUNK), jnp.int32),
          pltpu.VMEM((nchunk, CHUNK), jnp.int32),
          pltpu.VMEM((nchunk, CHUNK), jnp.int32),
          pltpu.VMEM((nchunk, neg_g, GATHER_W), jnp.int32),
          pltpu.VMEM((nchunk, n_neg, CHUNK), jnp.int32),
          pltpu.VMEM((CHUNK, pw), jnp.float32),
          pltpu.VMEM((CHUNK, pw), jnp.float32),
          pltpu.VMEM((neg_g, GATHER_W, pw), jnp.float32),
          pltpu.VMEM((nchunk, CHUNK), jnp.float32),
          pltpu.VMEM((nchunk, CHUNK), jnp.float32),
          pltpu.SemaphoreType.DMA,
      ],
  )
  pos_s, neg_s = sc(ug, uo, vpg, vpo, ngg, ngo, uw2, vw2)

  rows = 128
  total = pl.pallas_call(
      _tc_body,
      in_specs=[
          pl.BlockSpec((rows, b // rows), lambda: (0, 0)),
          pl.BlockSpec((rows, b // rows), lambda: (0, 0)),
      ],
      out_specs=pl.BlockSpec((1, 1), lambda: (0, 0)),
      out_shape=jax.ShapeDtypeStruct((1, 1), jnp.float32),
  )(pos_s.reshape(rows, b // rows), neg_s.reshape(rows, b // rows))

  return -total[0, 0] / batch_size


# speedup vs baseline: 14.8883x; 1.2133x over previous
import functools

import jax
import jax.numpy as jnp
from jax import lax
from jax.experimental import pallas as pl
from jax.experimental.pallas import tpu as pltpu
from jax.experimental.pallas import tpu_sc as plsc

NC = 2
NS = 16
NW = NC * NS
L = 16
CHUNK = 32
GATHER_W = 128


def _sc_body(nchunk, n_neg, dim,
             ug_h, vpg_h, ngg_h, u_w_h, v_w_h,
             pos_out_h, neg_out_h,
             idx_ug, idx_vpg, idx_ngg,
             u_rows, vp_rows, ng_rows,
             pos_part, neg_part, sem):
  neg_g = CHUNK * n_neg // GATHER_W
  nk = dim // L
  cid = lax.axis_index("c")
  sid = lax.axis_index("s")
  wid = sid * NC + cid

  pltpu.sync_copy(ug_h.at[wid], idx_ug)
  pltpu.sync_copy(vpg_h.at[wid], idx_vpg)
  pltpu.sync_copy(ngg_h.at[wid], idx_ngg)

  def chunk_body(c, carry):
    cp_u = pltpu.async_copy(u_w_h.at[idx_ug.at[c]], u_rows, sem)
    cp_vp = pltpu.async_copy(v_w_h.at[idx_vpg.at[c]], vp_rows, sem)
    cps = [
        pltpu.async_copy(v_w_h.at[idx_ngg.at[c, g]],
                         ng_rows.at[g], sem)
        for g in range(neg_g)
    ]
    cp_u.wait()
    cp_vp.wait()
    for cp in cps:
      cp.wait()

    def e_body(e, carry):
      base = e * n_neg
      pos = jnp.zeros((L,), jnp.float32)
      neg = jnp.zeros((L,), jnp.float32)
      for k in range(nk):
        u_k = u_rows[e, k * L:(k + 1) * L]
        vp_k = vp_rows[e, k * L:(k + 1) * L]
        p0 = base
        s = ng_rows[p0 // GATHER_W, p0 % GATHER_W, k * L:(k + 1) * L]
        for n in range(1, n_neg):
          p = base + n
          s = s + ng_rows[p // GATHER_W, p % GATHER_W, k * L:(k + 1) * L]
        pos = pos + u_k * vp_k
        neg = neg + u_k * s
      pos_part[e, :] = pos
      neg_part[e, :] = neg
      return carry

    lax.fori_loop(0, CHUNK, e_body, 0)
    pltpu.sync_copy(pos_part, pos_out_h.at[wid, c])
    pltpu.sync_copy(neg_part, neg_out_h.at[wid, c])
    return carry

  lax.fori_loop(0, nchunk, chunk_body, 0)


def _tc_body(pos_ref, neg_ref, out_ref):
  i = pl.program_id(0)
  p = jnp.sum(pos_ref[...], axis=1)
  n = jnp.sum(neg_ref[...], axis=1)
  t = jnp.sum(jax.nn.log_sigmoid(p) + jax.nn.log_sigmoid(-n))

  @pl.when(i == 0)
  def _init():
    out_ref[...] = jnp.zeros((1, 1), jnp.float32)

  out_ref[...] = out_ref[...] + t.reshape(1, 1)


@jax.jit
def kernel(u_pos, v_pos, v_neg, batch_size, u_weight, v_weight):
  b = u_pos.shape[0]
  n_neg = v_neg.shape[1]
  vocab, dim = u_weight.shape
  bpw = b // NW
  nchunk = bpw // CHUNK
  neg_g = CHUNK * n_neg // GATHER_W

  u_pos = u_pos.astype(jnp.int32)
  v_pos = v_pos.astype(jnp.int32)
  v_neg = v_neg.astype(jnp.int32)
  ug = u_pos.reshape(NW, nchunk, CHUNK)
  vpg = v_pos.reshape(NW, nchunk, CHUNK)
  ngg = v_neg.reshape(NW, nchunk, neg_g, GATHER_W)

  mesh = plsc.VectorSubcoreMesh(
      core_axis_name="c", subcore_axis_name="s",
      num_cores=NC, num_subcores=NS)
  part_ty = jax.ShapeDtypeStruct((NW, nchunk, CHUNK, L), jnp.float32)
  sc = pl.kernel(
      functools.partial(_sc_body, nchunk, n_neg, dim),
      out_type=(part_ty, part_ty),
      mesh=mesh,
      compiler_params=pltpu.CompilerParams(
          use_tc_tiling_on_sc=False, needs_layout_passes=False),
      scratch_types=[
          pltpu.VMEM((nchunk, CHUNK), jnp.int32),
          pltpu.VMEM((nchunk, CHUNK), jnp.int32),
          pltpu.VMEM((nchunk, neg_g, GATHER_W), jnp.int32),
          pltpu.VMEM((CHUNK, dim), jnp.float32),
          pltpu.VMEM((CHUNK, dim), jnp.float32),
          pltpu.VMEM((neg_g, GATHER_W, dim), jnp.float32),
          pltpu.VMEM((CHUNK, L), jnp.float32),
          pltpu.VMEM((CHUNK, L), jnp.float32),
          pltpu.SemaphoreType.DMA,
      ],
  )
  pos_s, neg_s = sc(ug, vpg, ngg, u_weight, v_weight)

  rows = 2048
  total = pl.pallas_call(
      _tc_body,
      grid=(b // rows,),
      in_specs=[
          pl.BlockSpec((rows, L), lambda i: (i, 0)),
          pl.BlockSpec((rows, L), lambda i: (i, 0)),
      ],
      out_specs=pl.BlockSpec((1, 1), lambda i: (0, 0)),
      out_shape=jax.ShapeDtypeStruct((1, 1), jnp.float32),
  )(pos_s.reshape(b, L), neg_s.reshape(b, L))

  return -total[0, 0] / batch_size


# speedup vs baseline: 15.2045x; 1.0212x over previous
import functools

import jax
import jax.numpy as jnp
from jax import lax
from jax.experimental import pallas as pl
from jax.experimental.pallas import tpu as pltpu
from jax.experimental.pallas import tpu_sc as plsc

NC = 2
NS = 16
NW = NC * NS
L = 16
CHUNK = 32
GATHER_W = 128


def _sc_body(nchunk, n_neg, dim,
             ug_h, vpg_h, ngg_h, u_w_h, v_w_h,
             pos_out_h, neg_out_h,
             idx_ug, idx_vpg, idx_ngg,
             u_rows, vp_rows, ng_rows,
             pos_part, neg_part, sem, sem2):
  neg_g = CHUNK * n_neg // GATHER_W
  nk = dim // L
  cid = lax.axis_index("c")
  sid = lax.axis_index("s")
  wid = sid * NC + cid

  pltpu.sync_copy(ug_h.at[wid], idx_ug)
  pltpu.sync_copy(vpg_h.at[wid], idx_vpg)
  pltpu.sync_copy(ngg_h.at[wid], idx_ngg)

  def issue(c, buf, sem):
    cps = [
        pltpu.async_copy(u_w_h.at[idx_ug.at[c]], u_rows.at[buf], sem),
        pltpu.async_copy(v_w_h.at[idx_vpg.at[c]], vp_rows.at[buf], sem),
    ]
    cps += [
        pltpu.async_copy(v_w_h.at[idx_ngg.at[c, g]],
                         ng_rows.at[buf, g], sem)
        for g in range(neg_g)
    ]
    return cps

  def compute(c, buf):
    def e_body(e, carry):
      base = e * n_neg
      pos = jnp.zeros((L,), jnp.float32)
      neg = jnp.zeros((L,), jnp.float32)
      for k in range(nk):
        u_k = u_rows[buf, e, k * L:(k + 1) * L]
        vp_k = vp_rows[buf, e, k * L:(k + 1) * L]
        p0 = base
        s = ng_rows[buf, p0 // GATHER_W, p0 % GATHER_W, k * L:(k + 1) * L]
        for n in range(1, n_neg):
          p = base + n
          s = s + ng_rows[buf, p // GATHER_W, p % GATHER_W,
                          k * L:(k + 1) * L]
        pos = pos + u_k * vp_k
        neg = neg + u_k * s
      pos_part[e, :] = pos
      neg_part[e, :] = neg
      return carry

    lax.fori_loop(0, CHUNK, e_body, 0)
    pltpu.sync_copy(pos_part, pos_out_h.at[wid, c])
    pltpu.sync_copy(neg_part, neg_out_h.at[wid, c])

  sems = (sem, sem2)
  pend = issue(0, 0, sems[0])
  for c in range(nchunk):
    nxt = issue(c + 1, (c + 1) % 2, sems[(c + 1) % 2]) if c + 1 < nchunk \
        else []
    for cp in pend:
      cp.wait()
    compute(c, c % 2)
    pend = nxt


def _tc_body(pos_ref, neg_ref, out_ref):
  i = pl.program_id(0)
  p = jnp.sum(pos_ref[...], axis=1)
  n = jnp.sum(neg_ref[...], axis=1)
  t = jnp.sum(jax.nn.log_sigmoid(p) + jax.nn.log_sigmoid(-n))

  @pl.when(i == 0)
  def _init():
    out_ref[...] = jnp.zeros((1, 1), jnp.float32)

  out_ref[...] = out_ref[...] + t.reshape(1, 1)


@jax.jit
def kernel(u_pos, v_pos, v_neg, batch_size, u_weight, v_weight):
  b = u_pos.shape[0]
  n_neg = v_neg.shape[1]
  vocab, dim = u_weight.shape
  bpw = b // NW
  nchunk = bpw // CHUNK
  neg_g = CHUNK * n_neg // GATHER_W

  u_pos = u_pos.astype(jnp.int32)
  v_pos = v_pos.astype(jnp.int32)
  v_neg = v_neg.astype(jnp.int32)
  ug = u_pos.reshape(NW, nchunk, CHUNK)
  vpg = v_pos.reshape(NW, nchunk, CHUNK)
  ngg = v_neg.reshape(NW, nchunk, neg_g, GATHER_W)

  mesh = plsc.VectorSubcoreMesh(
      core_axis_name="c", subcore_axis_name="s",
      num_cores=NC, num_subcores=NS)
  part_ty = jax.ShapeDtypeStruct((NW, nchunk, CHUNK, L), jnp.float32)
  sc = pl.kernel(
      functools.partial(_sc_body, nchunk, n_neg, dim),
      out_type=(part_ty, part_ty),
      mesh=mesh,
      compiler_params=pltpu.CompilerParams(
          use_tc_tiling_on_sc=False, needs_layout_passes=False),
      scratch_types=[
          pltpu.VMEM((nchunk, CHUNK), jnp.int32),
          pltpu.VMEM((nchunk, CHUNK), jnp.int32),
          pltpu.VMEM((nchunk, neg_g, GATHER_W), jnp.int32),
          pltpu.VMEM((2, CHUNK, dim), jnp.float32),
          pltpu.VMEM((2, CHUNK, dim), jnp.float32),
          pltpu.VMEM((2, neg_g, GATHER_W, dim), jnp.float32),
          pltpu.VMEM((CHUNK, L), jnp.float32),
          pltpu.VMEM((CHUNK, L), jnp.float32),
          pltpu.SemaphoreType.DMA,
          pltpu.SemaphoreType.DMA,
      ],
  )
  pos_s, neg_s = sc(ug, vpg, ngg, u_weight, v_weight)

  rows = 2048
  total = pl.pallas_call(
      _tc_body,
      grid=(b // rows,),
      in_specs=[
          pl.BlockSpec((rows, L), lambda i: (i, 0)),
          pl.BlockSpec((rows, L), lambda i: (i, 0)),
      ],
      out_specs=pl.BlockSpec((1, 1), lambda i: (0, 0)),
      out_shape=jax.ShapeDtypeStruct((1, 1), jnp.float32),
  )(pos_s.reshape(b, L), neg_s.reshape(b, L))

  return -total[0, 0] / batch_size
